# Initial kernel scaffold; baseline (speedup 1.0000x reference)
import jax
import jax.numpy as jnp
from jax.experimental import pallas as pl


def kernel(x, edge_index, W11, b11, W12, b12, W21, b21, W22, b22, W31, b31, W32, b32, Wa, ba, Wb, bb, Wc, bc, Wr, br, Wcls, bcls):
    raise NotImplementedError("write your pallas kernel here")


# speedup vs baseline: 4.8419x; 4.8419x over previous
import functools

import jax
import jax.numpy as jnp
from jax import lax
from jax.experimental import pallas as pl
from jax.experimental.pallas import tpu as pltpu
from jax.experimental.pallas import tpu_sc as plsc

N = 10000
E = 320000
D_IN = 128
H = 256
NC = 2
NS = 16
CHUNK = 128
NCH = E // CHUNK
ROWS_PER_SUB = 632


@functools.cache
def _sc_conv(feature_split):
  mesh = plsc.VectorSubcoreMesh(core_axis_name="c", subcore_axis_name="s")

  def body(table, src, dst, out, acc, srcbuf, dstbuf, rows, sem):
    c = lax.axis_index("c")
    s = lax.axis_index("s")
    off = c * N if feature_split else c * 0
    rbase = jnp.minimum(s * ROWS_PER_SUB, N - ROWS_PER_SUB)
    pltpu.sync_copy(
        table.at[pl.ds(off + rbase, ROWS_PER_SUB)],
        acc.at[pl.ds(rbase, ROWS_PER_SUB)],
    )
    plsc.subcore_barrier()

    if feature_split:
      first, stride = s, NS
    else:
      first, stride = s * NC + c, NS * NC
    nt = (NCH - first + stride - 1) // stride

    def step(t, carry):
      j = first + stride * t
      base = j * CHUNK
      pltpu.sync_copy(src.at[pl.ds(base, CHUNK)], srcbuf)
      pltpu.sync_copy(dst.at[pl.ds(base, CHUNK)], dstbuf)
      if feature_split:
        for k in range(CHUNK // 16):
          sl = pl.ds(k * 16, 16)
          srcbuf[sl] = srcbuf[sl] + off
      pltpu.async_copy(table.at[srcbuf], rows, sem).wait()
      pltpu.sync_copy(rows, acc.at[dstbuf], add=True)
      return carry

    lax.fori_loop(0, nt, step, 0)
    plsc.subcore_barrier()
    pltpu.sync_copy(
        acc.at[pl.ds(rbase, ROWS_PER_SUB)],
        out.at[pl.ds(c * N + rbase, ROWS_PER_SUB)],
    )

  return pl.kernel(
      body,
      out_type=jax.ShapeDtypeStruct((2 * N, 128), jnp.float32),
      mesh=mesh,
      scratch_types=[
          pltpu.VMEM_SHARED((N, 128), jnp.float32),
          pltpu.VMEM((CHUNK,), jnp.int32),
          pltpu.VMEM((CHUNK,), jnp.int32),
          pltpu.VMEM((CHUNK, 128), jnp.float32),
          pltpu.SemaphoreType.DMA,
      ],
  )


_B = 2000
_G = N // _B


def _mlp1_body(c1_ref, x_ref, w1_ref, b1_ref, w2_ref, b2_ref, o_ref):
  hp = c1_ref[0] + c1_ref[1] - x_ref[...]
  t = jnp.maximum(
      jnp.dot(hp, w1_ref[...], preferred_element_type=jnp.float32)
      + b1_ref[...], 0.0)
  y = jnp.maximum(
      jnp.dot(t, w2_ref[...], preferred_element_type=jnp.float32)
      + b2_ref[...], 0.0)
  o_ref[0] = y[:, :128]
  o_ref[1] = y[:, 128:]


def _mlp_mid_body(hp_ref, w1_ref, b1_ref, w2_ref, b2_ref, o_ref):
  t = jnp.maximum(
      jnp.dot(hp_ref[0], w1_ref[:128], preferred_element_type=jnp.float32)
      + jnp.dot(hp_ref[1], w1_ref[128:], preferred_element_type=jnp.float32)
      + b1_ref[...], 0.0)
  y = jnp.maximum(
      jnp.dot(t, w2_ref[...], preferred_element_type=jnp.float32)
      + b2_ref[...], 0.0)
  o_ref[0] = y[:, :128]
  o_ref[1] = y[:, 128:]


def _attn_body(hp_ref, w1_ref, b1_ref, w2_ref, b2_ref,
               wa_ref, ba_ref, wb_ref, bb_ref, wc_ref, bc_ref,
               wr_ref, br_ref, wcls_ref, bcls_ref,
               logits_ref, hz_ref, s_out_ref,
               m_ref, sum_ref, v_ref):
  i = pl.program_id(0)

  @pl.when(i == 0)
  def _():
    m_ref[0, 0] = -jnp.inf
    sum_ref[0, 0] = 0.0
    v_ref[...] = jnp.zeros_like(v_ref)

  t = jnp.maximum(
      jnp.dot(hp_ref[0], w1_ref[:128], preferred_element_type=jnp.float32)
      + jnp.dot(hp_ref[1], w1_ref[128:], preferred_element_type=jnp.float32)
      + b1_ref[...], 0.0)
  x3 = jnp.maximum(
      jnp.dot(t, w2_ref[...], preferred_element_type=jnp.float32)
      + b2_ref[...], 0.0)
  a = jnp.tanh(
      jnp.dot(x3, wa_ref[...], preferred_element_type=jnp.float32)
      + ba_ref[...])
  b = jax.nn.sigmoid(
      jnp.dot(x3, wb_ref[...], preferred_element_type=jnp.float32)
      + bb_ref[...])
  att = (jnp.dot(a * b, wc_ref[...], preferred_element_type=jnp.float32)
         + bc_ref[...])

  m_old = m_ref[0, 0]
  m_new = jnp.maximum(m_old, jnp.max(att))
  corr = jnp.exp(m_old - m_new)
  w = jnp.exp(att - m_new)
  sum_ref[0, 0] = sum_ref[0, 0] * corr + jnp.sum(w)
  v_ref[...] = v_ref[...] * corr + jnp.sum(w * x3, axis=0, keepdims=True)
  m_ref[0, 0] = m_new

  @pl.when(i == _G - 1)
  def _():
    h = v_ref[...] / sum_ref[0, 0]
    h2 = jnp.maximum(
        jnp.dot(h, wr_ref[...], preferred_element_type=jnp.float32)
        + br_ref[...], 0.0)
    logits = (jnp.dot(h2, wcls_ref[...], preferred_element_type=jnp.float32)
              + bcls_ref[...])
    hz = jax.nn.sigmoid(logits)
    om = 1.0 - hz
    s0 = om[:, 0:1]
    s1 = s0 * om[:, 1:2]
    s2 = s1 * om[:, 2:3]
    s3 = s2 * om[:, 3:4]
    logits_ref[...] = logits
    hz_ref[...] = hz
    s_out_ref[...] = jnp.concatenate([s0, s1, s2, s3], axis=1)


def _full_spec(shape):
  return pl.BlockSpec(shape, lambda i: tuple(0 for _ in shape))


def _mlp1(c1, x, w1, b1, w2, b2):
  return pl.pallas_call(
      _mlp1_body,
      grid=(_G,),
      in_specs=[
          pl.BlockSpec((2, _B, 128), lambda i: (0, i, 0)),
          pl.BlockSpec((_B, 128), lambda i: (i, 0)),
          _full_spec((128, H)),
          _full_spec((1, H)),
          _full_spec((H, H)),
          _full_spec((1, H)),
      ],
      out_specs=pl.BlockSpec((2, _B, 128), lambda i: (0, i, 0)),
      out_shape=jax.ShapeDtypeStruct((2, N, 128), jnp.float32),
      compiler_params=pltpu.CompilerParams(
          dimension_semantics=("arbitrary",)),
  )(c1, x, w1, b1, w2, b2)


def _mlp_mid(hp, w1, b1, w2, b2):
  return pl.pallas_call(
      _mlp_mid_body,
      grid=(_G,),
      in_specs=[
          pl.BlockSpec((2, _B, 128), lambda i: (0, i, 0)),
          _full_spec((H, H)),
          _full_spec((1, H)),
          _full_spec((H, H)),
          _full_spec((1, H)),
      ],
      out_specs=pl.BlockSpec((2, _B, 128), lambda i: (0, i, 0)),
      out_shape=jax.ShapeDtypeStruct((2, N, 128), jnp.float32),
      compiler_params=pltpu.CompilerParams(
          dimension_semantics=("arbitrary",)),
  )(hp, w1, b1, w2, b2)


def _attn(hp, w1, b1, w2, b2, wa, ba, wb, bb, wc, bc, wr, br, wcls, bcls):
  C = wcls.shape[1]
  return pl.pallas_call(
      _attn_body,
      grid=(_G,),
      in_specs=[
          pl.BlockSpec((2, _B, 128), lambda i: (0, i, 0)),
          _full_spec((H, H)), _full_spec((1, H)),
          _full_spec((H, H)), _full_spec((1, H)),
          _full_spec((H, H)), _full_spec((1, H)),
          _full_spec((H, H)), _full_spec((1, H)),
          _full_spec((H, 1)), _full_spec((1, 1)),
          _full_spec((H, H)), _full_spec((1, H)),
          _full_spec((H, C)), _full_spec((1, C)),
      ],
      out_specs=[
          pl.BlockSpec((1, C), lambda i: (0, 0)),
          pl.BlockSpec((1, C), lambda i: (0, 0)),
          pl.BlockSpec((1, C), lambda i: (0, 0)),
      ],
      out_shape=[
          jax.ShapeDtypeStruct((1, C), jnp.float32),
          jax.ShapeDtypeStruct((1, C), jnp.float32),
          jax.ShapeDtypeStruct((1, C), jnp.float32),
      ],
      scratch_shapes=[
          pltpu.SMEM((1, 1), jnp.float32),
          pltpu.SMEM((1, 1), jnp.float32),
          pltpu.VMEM((1, H), jnp.float32),
      ],
      compiler_params=pltpu.CompilerParams(
          dimension_semantics=("arbitrary",)),
  )(hp, w1, b1, w2, b2, wa, ba, wb, bb, wc, bc, wr, br, wcls, bcls)


def kernel(x, edge_index, W11, b11, W12, b12, W21, b21, W22, b22,
           W31, b31, W32, b32, Wa, ba, Wb, bb, Wc, bc, Wr, br, Wcls, bcls):
  src = edge_index[0]
  dst = edge_index[1]
  r = lambda v: v.reshape(1, -1)

  c1 = _sc_conv(False)(x, src, dst)
  y1 = _mlp1(c1.reshape(2, N, 128), x, W11, r(b11), W12, r(b12))
  c2 = _sc_conv(True)(y1.reshape(2 * N, 128), src, dst)
  y2 = _mlp_mid(c2.reshape(2, N, 128), W21, r(b21), W22, r(b22))
  c3 = _sc_conv(True)(y2.reshape(2 * N, 128), src, dst)
  logits, hazards, S = _attn(
      c3.reshape(2, N, 128), W31, r(b31), W32, r(b32),
      Wa, r(ba), Wb, r(bb), Wc, r(bc), Wr, r(br), Wcls, r(bcls))
  return (logits, hazards, S)


# speedup vs baseline: 7.8747x; 1.6264x over previous
import functools

import jax
import jax.numpy as jnp
from jax import lax
from jax.experimental import pallas as pl
from jax.experimental.pallas import tpu as pltpu
from jax.experimental.pallas import tpu_sc as plsc

N = 10000
E = 320000
D_IN = 128
H = 256
NC = 2
NS = 16
CHUNK = 128
NCH = E // CHUNK
ROWS_PER_SUB = 632


@functools.cache
def _sc_conv(feature_split):
  mesh = plsc.VectorSubcoreMesh(core_axis_name="c", subcore_axis_name="s")

  def body(table, src, dst, out, acc,
           srcbuf0, dstbuf0, rows0, sem0,
           srcbuf1, dstbuf1, rows1, sem1):
    c = lax.axis_index("c")
    s = lax.axis_index("s")
    off = c * N if feature_split else c * 0
    rbase = jnp.minimum(s * ROWS_PER_SUB, N - ROWS_PER_SUB)
    pltpu.sync_copy(
        table.at[pl.ds(off + rbase, ROWS_PER_SUB)],
        acc.at[pl.ds(rbase, ROWS_PER_SUB)],
    )
    plsc.subcore_barrier()

    if feature_split:
      first, stride = s, NS
    else:
      first, stride = s * NC + c, NS * NC
    nt = (NCH - first + stride - 1) // stride

    bufs = ((srcbuf0, dstbuf0, rows0, sem0), (srcbuf1, dstbuf1, rows1, sem1))

    def load_and_fire(b, t):
      sb, db, rw, sm = bufs[b]
      base = (first + stride * t) * CHUNK
      pltpu.sync_copy(src.at[pl.ds(base, CHUNK)], sb)
      pltpu.sync_copy(dst.at[pl.ds(base, CHUNK)], db)
      if feature_split:
        for k in range(CHUNK // 16):
          sl = pl.ds(k * 16, 16)
          sb[sl] = sb[sl] + off
      pltpu.async_copy(table.at[sb], rw, sm)

    def drain_and_scatter(b):
      sb, db, rw, sm = bufs[b]
      pltpu.make_async_copy(table.at[pl.ds(0, CHUNK)], rw, sm).wait()
      pltpu.sync_copy(rw, acc.at[db], add=True)

    load_and_fire(0, 0)

    def pair(t2, carry):
      for b in range(2):
        t = 2 * t2 + b

        @pl.when(t + 1 < nt)
        def _():
          load_and_fire(1 - b, t + 1)

        drain_and_scatter(b)
      return carry

    lax.fori_loop(0, nt // 2, pair, 0)

    @pl.when(nt % 2 == 1)
    def _():
      drain_and_scatter(0)

    plsc.subcore_barrier()
    pltpu.sync_copy(
        acc.at[pl.ds(rbase, ROWS_PER_SUB)],
        out.at[pl.ds(c * N + rbase, ROWS_PER_SUB)],
    )

  return pl.kernel(
      body,
      out_type=jax.ShapeDtypeStruct((2 * N, 128), jnp.float32),
      mesh=mesh,
      scratch_types=[
          pltpu.VMEM_SHARED((N, 128), jnp.float32),
          pltpu.VMEM((CHUNK,), jnp.int32),
          pltpu.VMEM((CHUNK,), jnp.int32),
          pltpu.VMEM((CHUNK, 128), jnp.float32),
          pltpu.SemaphoreType.DMA,
          pltpu.VMEM((CHUNK,), jnp.int32),
          pltpu.VMEM((CHUNK,), jnp.int32),
          pltpu.VMEM((CHUNK, 128), jnp.float32),
          pltpu.SemaphoreType.DMA,
      ],
  )


_B = 2000
_G = N // _B


def _mlp1_body(c1_ref, x_ref, w1_ref, b1_ref, w2_ref, b2_ref, o_ref):
  hp = c1_ref[0] + c1_ref[1] - x_ref[...]
  t = jnp.maximum(
      jnp.dot(hp, w1_ref[...], preferred_element_type=jnp.float32)
      + b1_ref[...], 0.0)
  y = jnp.maximum(
      jnp.dot(t, w2_ref[...], preferred_element_type=jnp.float32)
      + b2_ref[...], 0.0)
  o_ref[0] = y[:, :128]
  o_ref[1] = y[:, 128:]


def _mlp_mid_body(hp_ref, w1_ref, b1_ref, w2_ref, b2_ref, o_ref):
  t = jnp.maximum(
      jnp.dot(hp_ref[0], w1_ref[:128], preferred_element_type=jnp.float32)
      + jnp.dot(hp_ref[1], w1_ref[128:], preferred_element_type=jnp.float32)
      + b1_ref[...], 0.0)
  y = jnp.maximum(
      jnp.dot(t, w2_ref[...], preferred_element_type=jnp.float32)
      + b2_ref[...], 0.0)
  o_ref[0] = y[:, :128]
  o_ref[1] = y[:, 128:]


def _attn_body(hp_ref, w1_ref, b1_ref, w2_ref, b2_ref,
               wa_ref, ba_ref, wb_ref, bb_ref, wc_ref, bc_ref,
               wr_ref, br_ref, wcls_ref, bcls_ref,
               logits_ref, hz_ref, s_out_ref,
               m_ref, sum_ref, v_ref):
  i = pl.program_id(0)

  @pl.when(i == 0)
  def _():
    m_ref[0, 0] = -jnp.inf
    sum_ref[0, 0] = 0.0
    v_ref[...] = jnp.zeros_like(v_ref)

  t = jnp.maximum(
      jnp.dot(hp_ref[0], w1_ref[:128], preferred_element_type=jnp.float32)
      + jnp.dot(hp_ref[1], w1_ref[128:], preferred_element_type=jnp.float32)
      + b1_ref[...], 0.0)
  x3 = jnp.maximum(
      jnp.dot(t, w2_ref[...], preferred_element_type=jnp.float32)
      + b2_ref[...], 0.0)
  a = jnp.tanh(
      jnp.dot(x3, wa_ref[...], preferred_element_type=jnp.float32)
      + ba_ref[...])
  b = jax.nn.sigmoid(
      jnp.dot(x3, wb_ref[...], preferred_element_type=jnp.float32)
      + bb_ref[...])
  att = (jnp.dot(a * b, wc_ref[...], preferred_element_type=jnp.float32)
         + bc_ref[...])

  m_old = m_ref[0, 0]
  m_new = jnp.maximum(m_old, jnp.max(att))
  corr = jnp.exp(m_old - m_new)
  w = jnp.exp(att - m_new)
  sum_ref[0, 0] = sum_ref[0, 0] * corr + jnp.sum(w)
  v_ref[...] = v_ref[...] * corr + jnp.sum(w * x3, axis=0, keepdims=True)
  m_ref[0, 0] = m_new

  @pl.when(i == _G - 1)
  def _():
    h = v_ref[...] / sum_ref[0, 0]
    h2 = jnp.maximum(
        jnp.dot(h, wr_ref[...], preferred_element_type=jnp.float32)
        + br_ref[...], 0.0)
    logits = (jnp.dot(h2, wcls_ref[...], preferred_element_type=jnp.float32)
              + bcls_ref[...])
    hz = jax.nn.sigmoid(logits)
    om = 1.0 - hz
    s0 = om[:, 0:1]
    s1 = s0 * om[:, 1:2]
    s2 = s1 * om[:, 2:3]
    s3 = s2 * om[:, 3:4]
    logits_ref[...] = logits
    hz_ref[...] = hz
    s_out_ref[...] = jnp.concatenate([s0, s1, s2, s3], axis=1)


def _full_spec(shape):
  return pl.BlockSpec(shape, lambda i: tuple(0 for _ in shape))


def _mlp1(c1, x, w1, b1, w2, b2):
  return pl.pallas_call(
      _mlp1_body,
      grid=(_G,),
      in_specs=[
          pl.BlockSpec((2, _B, 128), lambda i: (0, i, 0)),
          pl.BlockSpec((_B, 128), lambda i: (i, 0)),
          _full_spec((128, H)),
          _full_spec((1, H)),
          _full_spec((H, H)),
          _full_spec((1, H)),
      ],
      out_specs=pl.BlockSpec((2, _B, 128), lambda i: (0, i, 0)),
      out_shape=jax.ShapeDtypeStruct((2, N, 128), jnp.float32),
      compiler_params=pltpu.CompilerParams(
          dimension_semantics=("arbitrary",)),
  )(c1, x, w1, b1, w2, b2)


def _mlp_mid(hp, w1, b1, w2, b2):
  return pl.pallas_call(
      _mlp_mid_body,
      grid=(_G,),
      in_specs=[
          pl.BlockSpec((2, _B, 128), lambda i: (0, i, 0)),
          _full_spec((H, H)),
          _full_spec((1, H)),
          _full_spec((H, H)),
          _full_spec((1, H)),
      ],
      out_specs=pl.BlockSpec((2, _B, 128), lambda i: (0, i, 0)),
      out_shape=jax.ShapeDtypeStruct((2, N, 128), jnp.float32),
      compiler_params=pltpu.CompilerParams(
          dimension_semantics=("arbitrary",)),
  )(hp, w1, b1, w2, b2)


def _attn(hp, w1, b1, w2, b2, wa, ba, wb, bb, wc, bc, wr, br, wcls, bcls):
  C = wcls.shape[1]
  return pl.pallas_call(
      _attn_body,
      grid=(_G,),
      in_specs=[
          pl.BlockSpec((2, _B, 128), lambda i: (0, i, 0)),
          _full_spec((H, H)), _full_spec((1, H)),
          _full_spec((H, H)), _full_spec((1, H)),
          _full_spec((H, H)), _full_spec((1, H)),
          _full_spec((H, H)), _full_spec((1, H)),
          _full_spec((H, 1)), _full_spec((1, 1)),
          _full_spec((H, H)), _full_spec((1, H)),
          _full_spec((H, C)), _full_spec((1, C)),
      ],
      out_specs=[
          pl.BlockSpec((1, C), lambda i: (0, 0)),
          pl.BlockSpec((1, C), lambda i: (0, 0)),
          pl.BlockSpec((1, C), lambda i: (0, 0)),
      ],
      out_shape=[
          jax.ShapeDtypeStruct((1, C), jnp.float32),
          jax.ShapeDtypeStruct((1, C), jnp.float32),
          jax.ShapeDtypeStruct((1, C), jnp.float32),
      ],
      scratch_shapes=[
          pltpu.SMEM((1, 1), jnp.float32),
          pltpu.SMEM((1, 1), jnp.float32),
          pltpu.VMEM((1, H), jnp.float32),
      ],
      compiler_params=pltpu.CompilerParams(
          dimension_semantics=("arbitrary",)),
  )(hp, w1, b1, w2, b2, wa, ba, wb, bb, wc, bc, wr, br, wcls, bcls)


def kernel(x, edge_index, W11, b11, W12, b12, W21, b21, W22, b22,
           W31, b31, W32, b32, Wa, ba, Wb, bb, Wc, bc, Wr, br, Wcls, bcls):
  src = edge_index[0]
  dst = edge_index[1]
  r = lambda v: v.reshape(1, -1)

  c1 = _sc_conv(False)(x, src, dst)
  y1 = _mlp1(c1.reshape(2, N, 128), x, W11, r(b11), W12, r(b12))
  c2 = _sc_conv(True)(y1.reshape(2 * N, 128), src, dst)
  y2 = _mlp_mid(c2.reshape(2, N, 128), W21, r(b21), W22, r(b22))
  c3 = _sc_conv(True)(y2.reshape(2 * N, 128), src, dst)
  logits, hazards, S = _attn(
      c3.reshape(2, N, 128), W31, r(b31), W32, r(b32),
      Wa, r(ba), Wb, r(bb), Wc, r(bc), Wr, r(br), Wcls, r(bcls))
  return (logits, hazards, S)


# speedup vs baseline: 8.9464x; 1.1361x over previous
import functools

import jax
import jax.numpy as jnp
from jax import lax
from jax.experimental import pallas as pl
from jax.experimental.pallas import tpu as pltpu
from jax.experimental.pallas import tpu_sc as plsc

N = 10000
E = 320000
D_IN = 128
H = 256
NC = 2
NS = 16
CHUNK = 128
NCH = E // CHUNK
ROWS_PER_SUB = 632


@functools.cache
def _sc_conv(feature_split):
  mesh = plsc.VectorSubcoreMesh(core_axis_name="c", subcore_axis_name="s")

  U = NCH // NS if feature_split else NCH // (NS * NC)
  U -= U % 6
  NTAIL = NCH - U * (NS if feature_split else NS * NC)
  U6 = U // 6

  def body(table, src, dst, out, acc,
           sb0, sb1, sb2, db0, db1, db2, rows0, rows1,
           is0, is1, is2, gs0, gs1, ss0, ss1):
    c = lax.axis_index("c")
    s = lax.axis_index("s")
    off = c * N if feature_split else c * 0
    rbase = jnp.minimum(s * ROWS_PER_SUB, N - ROWS_PER_SUB)
    pltpu.sync_copy(
        table.at[pl.ds(off + rbase, ROWS_PER_SUB)],
        acc.at[pl.ds(rbase, ROWS_PER_SUB)],
    )
    plsc.subcore_barrier()

    wid = s if feature_split else s * NC + c
    cb = wid * U
    sb = (sb0, sb1, sb2)
    db = (db0, db1, db2)
    isem = (is0, is1, is2)
    rows = (rows0, rows1)
    gsem = (gs0, gs1)
    ssem = (ss0, ss1)

    def fire_idx(q, u):
      base = (cb + u) * CHUNK
      pltpu.async_copy(src.at[pl.ds(base, CHUNK)], sb[q], isem[q])
      pltpu.async_copy(dst.at[pl.ds(base, CHUNK)], db[q], isem[q])

    def wait_idx(q):
      pltpu.make_async_copy(src.at[pl.ds(0, CHUNK)], sb[q], isem[q]).wait()
      pltpu.make_async_copy(dst.at[pl.ds(0, CHUNK)], db[q], isem[q]).wait()
      if feature_split:
        for k in range(CHUNK // 16):
          sl = pl.ds(k * 16, 16)
          sb[q][sl] = sb[q][sl] + off

    def fire_gather(q, p):
      pltpu.async_copy(table.at[sb[q]], rows[p], gsem[p])

    def wait_gather(p):
      pltpu.make_async_copy(table.at[pl.ds(0, CHUNK)], rows[p], gsem[p]).wait()

    def fire_scatter(q, p):
      pltpu.async_copy(rows[p], acc.at[db[q]], ssem[p], add=True)

    def wait_scatter(p):
      pltpu.make_async_copy(table.at[pl.ds(0, CHUNK)], rows[p], ssem[p]).wait()

    fire_idx(0, 0)
    fire_idx(1, 1)
    wait_idx(0)
    fire_gather(0, 0)

    def block(t, carry):
      u0 = 6 * t
      for l in range(6):
        u = u0 + l
        q, p = l % 3, l % 2
        wait_gather(p)
        fire_scatter(q, p)
        if l == 0:
          @pl.when(t > 0)
          def _():
            wait_scatter(1 - p)
        else:
          wait_scatter(1 - p)
        if l in (4, 5):
          @pl.when(t < U6 - 1)
          def _():
            fire_idx((l + 2) % 3, u + 2)
        else:
          fire_idx((l + 2) % 3, u + 2)
        if l == 5:
          @pl.when(t < U6 - 1)
          def _():
            wait_idx(0)
            fire_gather(0, 1 - p)
        else:
          wait_idx((l + 1) % 3)
          fire_gather((l + 1) % 3, 1 - p)
      return carry

    lax.fori_loop(0, U6, block, 0)
    wait_scatter(1)

    @pl.when(wid < NTAIL)
    def _():
      base = (NCH - NTAIL + wid) * CHUNK
      pltpu.sync_copy(src.at[pl.ds(base, CHUNK)], sb0)
      pltpu.sync_copy(dst.at[pl.ds(base, CHUNK)], db0)
      if feature_split:
        for k in range(CHUNK // 16):
          sl = pl.ds(k * 16, 16)
          sb0[sl] = sb0[sl] + off
      pltpu.async_copy(table.at[sb0], rows0, gs0).wait()
      pltpu.sync_copy(rows0, acc.at[db0], add=True)

    plsc.subcore_barrier()
    pltpu.sync_copy(
        acc.at[pl.ds(rbase, ROWS_PER_SUB)],
        out.at[pl.ds(c * N + rbase, ROWS_PER_SUB)],
    )

  return pl.kernel(
      body,
      out_type=jax.ShapeDtypeStruct((2 * N, 128), jnp.float32),
      mesh=mesh,
      scratch_types=(
          [pltpu.VMEM_SHARED((N, 128), jnp.float32)]
          + [pltpu.VMEM((CHUNK,), jnp.int32)] * 6
          + [pltpu.VMEM((CHUNK, 128), jnp.float32)] * 2
          + [pltpu.SemaphoreType.DMA] * 7
      ),
  )


_B = 2000
_G = N // _B


def _mlp1_body(c1_ref, x_ref, w1_ref, b1_ref, w2_ref, b2_ref, o_ref):
  hp = c1_ref[0] + c1_ref[1] - x_ref[...]
  t = jnp.maximum(
      jnp.dot(hp, w1_ref[...], preferred_element_type=jnp.float32)
      + b1_ref[...], 0.0)
  y = jnp.maximum(
      jnp.dot(t, w2_ref[...], preferred_element_type=jnp.float32)
      + b2_ref[...], 0.0)
  o_ref[0] = y[:, :128]
  o_ref[1] = y[:, 128:]


def _mlp_mid_body(hp_ref, w1_ref, b1_ref, w2_ref, b2_ref, o_ref):
  t = jnp.maximum(
      jnp.dot(hp_ref[0], w1_ref[:128], preferred_element_type=jnp.float32)
      + jnp.dot(hp_ref[1], w1_ref[128:], preferred_element_type=jnp.float32)
      + b1_ref[...], 0.0)
  y = jnp.maximum(
      jnp.dot(t, w2_ref[...], preferred_element_type=jnp.float32)
      + b2_ref[...], 0.0)
  o_ref[0] = y[:, :128]
  o_ref[1] = y[:, 128:]


def _attn_body(hp_ref, w1_ref, b1_ref, w2_ref, b2_ref,
               wa_ref, ba_ref, wb_ref, bb_ref, wc_ref, bc_ref,
               wr_ref, br_ref, wcls_ref, bcls_ref,
               logits_ref, hz_ref, s_out_ref,
               m_ref, sum_ref, v_ref):
  i = pl.program_id(0)

  @pl.when(i == 0)
  def _():
    m_ref[0, 0] = -jnp.inf
    sum_ref[0, 0] = 0.0
    v_ref[...] = jnp.zeros_like(v_ref)

  t = jnp.maximum(
      jnp.dot(hp_ref[0], w1_ref[:128], preferred_element_type=jnp.float32)
      + jnp.dot(hp_ref[1], w1_ref[128:], preferred_element_type=jnp.float32)
      + b1_ref[...], 0.0)
  x3 = jnp.maximum(
      jnp.dot(t, w2_ref[...], preferred_element_type=jnp.float32)
      + b2_ref[...], 0.0)
  a = jnp.tanh(
      jnp.dot(x3, wa_ref[...], preferred_element_type=jnp.float32)
      + ba_ref[...])
  b = jax.nn.sigmoid(
      jnp.dot(x3, wb_ref[...], preferred_element_type=jnp.float32)
      + bb_ref[...])
  att = (jnp.dot(a * b, wc_ref[...], preferred_element_type=jnp.float32)
         + bc_ref[...])

  m_old = m_ref[0, 0]
  m_new = jnp.maximum(m_old, jnp.max(att))
  corr = jnp.exp(m_old - m_new)
  w = jnp.exp(att - m_new)
  sum_ref[0, 0] = sum_ref[0, 0] * corr + jnp.sum(w)
  v_ref[...] = v_ref[...] * corr + jnp.sum(w * x3, axis=0, keepdims=True)
  m_ref[0, 0] = m_new

  @pl.when(i == _G - 1)
  def _():
    h = v_ref[...] / sum_ref[0, 0]
    h2 = jnp.maximum(
        jnp.dot(h, wr_ref[...], preferred_element_type=jnp.float32)
        + br_ref[...], 0.0)
    logits = (jnp.dot(h2, wcls_ref[...], preferred_element_type=jnp.float32)
              + bcls_ref[...])
    hz = jax.nn.sigmoid(logits)
    om = 1.0 - hz
    s0 = om[:, 0:1]
    s1 = s0 * om[:, 1:2]
    s2 = s1 * om[:, 2:3]
    s3 = s2 * om[:, 3:4]
    logits_ref[...] = logits
    hz_ref[...] = hz
    s_out_ref[...] = jnp.concatenate([s0, s1, s2, s3], axis=1)


def _full_spec(shape):
  return pl.BlockSpec(shape, lambda i: tuple(0 for _ in shape))


def _mlp1(c1, x, w1, b1, w2, b2):
  return pl.pallas_call(
      _mlp1_body,
      grid=(_G,),
      in_specs=[
          pl.BlockSpec((2, _B, 128), lambda i: (0, i, 0)),
          pl.BlockSpec((_B, 128), lambda i: (i, 0)),
          _full_spec((128, H)),
          _full_spec((1, H)),
          _full_spec((H, H)),
          _full_spec((1, H)),
      ],
      out_specs=pl.BlockSpec((2, _B, 128), lambda i: (0, i, 0)),
      out_shape=jax.ShapeDtypeStruct((2, N, 128), jnp.float32),
      compiler_params=pltpu.CompilerParams(
          dimension_semantics=("arbitrary",)),
  )(c1, x, w1, b1, w2, b2)


def _mlp_mid(hp, w1, b1, w2, b2):
  return pl.pallas_call(
      _mlp_mid_body,
      grid=(_G,),
      in_specs=[
          pl.BlockSpec((2, _B, 128), lambda i: (0, i, 0)),
          _full_spec((H, H)),
          _full_spec((1, H)),
          _full_spec((H, H)),
          _full_spec((1, H)),
      ],
      out_specs=pl.BlockSpec((2, _B, 128), lambda i: (0, i, 0)),
      out_shape=jax.ShapeDtypeStruct((2, N, 128), jnp.float32),
      compiler_params=pltpu.CompilerParams(
          dimension_semantics=("arbitrary",)),
  )(hp, w1, b1, w2, b2)


def _attn(hp, w1, b1, w2, b2, wa, ba, wb, bb, wc, bc, wr, br, wcls, bcls):
  C = wcls.shape[1]
  return pl.pallas_call(
      _attn_body,
      grid=(_G,),
      in_specs=[
          pl.BlockSpec((2, _B, 128), lambda i: (0, i, 0)),
          _full_spec((H, H)), _full_spec((1, H)),
          _full_spec((H, H)), _full_spec((1, H)),
          _full_spec((H, H)), _full_spec((1, H)),
          _full_spec((H, H)), _full_spec((1, H)),
          _full_spec((H, 1)), _full_spec((1, 1)),
          _full_spec((H, H)), _full_spec((1, H)),
          _full_spec((H, C)), _full_spec((1, C)),
      ],
      out_specs=[
          pl.BlockSpec((1, C), lambda i: (0, 0)),
          pl.BlockSpec((1, C), lambda i: (0, 0)),
          pl.BlockSpec((1, C), lambda i: (0, 0)),
      ],
      out_shape=[
          jax.ShapeDtypeStruct((1, C), jnp.float32),
          jax.ShapeDtypeStruct((1, C), jnp.float32),
          jax.ShapeDtypeStruct((1, C), jnp.float32),
      ],
      scratch_shapes=[
          pltpu.SMEM((1, 1), jnp.float32),
          pltpu.SMEM((1, 1), jnp.float32),
          pltpu.VMEM((1, H), jnp.float32),
      ],
      compiler_params=pltpu.CompilerParams(
          dimension_semantics=("arbitrary",)),
  )(hp, w1, b1, w2, b2, wa, ba, wb, bb, wc, bc, wr, br, wcls, bcls)


def kernel(x, edge_index, W11, b11, W12, b12, W21, b21, W22, b22,
           W31, b31, W32, b32, Wa, ba, Wb, bb, Wc, bc, Wr, br, Wcls, bcls):
  src = edge_index[0]
  dst = edge_index[1]
  r = lambda v: v.reshape(1, -1)

  c1 = _sc_conv(False)(x, src, dst)
  y1 = _mlp1(c1.reshape(2, N, 128), x, W11, r(b11), W12, r(b12))
  c2 = _sc_conv(True)(y1.reshape(2 * N, 128), src, dst)
  y2 = _mlp_mid(c2.reshape(2, N, 128), W21, r(b21), W22, r(b22))
  c3 = _sc_conv(True)(y2.reshape(2 * N, 128), src, dst)
  logits, hazards, S = _attn(
      c3.reshape(2, N, 128), W31, r(b31), W32, r(b32),
      Wa, r(ba), Wb, r(bb), Wc, r(bc), Wr, r(br), Wcls, r(bcls))
  return (logits, hazards, S)
